# Initial kernel scaffold; baseline (speedup 1.0000x reference)
import jax
import jax.numpy as jnp
from jax.experimental import pallas as pl


def kernel(x, A, X, W_x, W_ea, W_t1, W_t2):
    raise NotImplementedError("write your pallas kernel here")


# speedup vs baseline: 1.4377x; 1.4377x over previous
import functools

import jax
import jax.numpy as jnp
from jax import lax
from jax.experimental import pallas as pl
from jax.experimental.pallas import tpu as pltpu
from jax.experimental.pallas import tpu_sc as plsc

HID = 128
N_NODES = 10000
N_EDGES = 320000

NC, NS = 2, 16
NW = NC * NS
CHUNK = 80

EDGE_PER_W = N_EDGES // NW
EDGE_CHUNKS = EDGE_PER_W // CHUNK
NODE_WORKERS = 25
NODE_PER_W = N_NODES // NODE_WORKERS
NODE_CHUNKS = NODE_PER_W // CHUNK
LANES = 16


def _fuse_tables(W_t1, W_t2):
    def body(w1_ref, w2_ref, out_ref):
        w1 = w1_ref[...]
        w2 = w2_ref[...]
        out_ref[...] = w1[:, None, :] + w2[None, :, :]

    T = pl.pallas_call(
        body,
        out_shape=jax.ShapeDtypeStruct((16, 16, HID), jnp.float32),
    )(W_t1, W_t2)
    return T.reshape(16 * 16, HID)


def _sc_gather(x, A, Xa, Xb, W_x, W_ea, T):
    mesh = plsc.VectorSubcoreMesh(core_axis_name="c", subcore_axis_name="s")

    @functools.partial(
        pl.kernel,
        out_type=(
            jax.ShapeDtypeStruct((N_NODES, HID), jnp.float32),
            jax.ShapeDtypeStruct((N_EDGES, HID), jnp.float32),
            jax.ShapeDtypeStruct((N_EDGES, HID), jnp.float32),
        ),
        mesh=mesh,
        scratch_types=[
            pltpu.VMEM((CHUNK,), jnp.int32),
            pltpu.VMEM((CHUNK,), jnp.int32),
            pltpu.VMEM((CHUNK, HID), jnp.float32),
            pltpu.SemaphoreType.DMA,
        ],
    )
    def k(x_hbm, A_hbm, Xa_hbm, Xb_hbm, Wx_hbm, Wea_hbm, T_hbm,
          out_x, out_A, out_X, idx_v, idx2_v, rows_v, sem):
        wid = lax.axis_index("s") * NC + lax.axis_index("c")
        ebase = wid * EDGE_PER_W

        def a_body(i, carry):
            base = ebase + i * CHUNK
            pltpu.sync_copy(A_hbm.at[pl.ds(base, CHUNK)], idx_v)
            pltpu.async_copy(Wea_hbm.at[idx_v], rows_v, sem).wait()
            pltpu.sync_copy(rows_v, out_A.at[pl.ds(base, CHUNK)])
            return carry

        lax.fori_loop(0, EDGE_CHUNKS, a_body, 0)

        def x_body(i, carry):
            base = ebase + i * CHUNK
            pltpu.sync_copy(Xa_hbm.at[pl.ds(base, CHUNK)], idx_v)
            pltpu.sync_copy(Xb_hbm.at[pl.ds(base, CHUNK)], idx2_v)
            for j in range(CHUNK // LANES):
                sl = pl.ds(j * LANES, LANES)
                idx_v[sl] = idx_v[sl] * 16 + idx2_v[sl]
            pltpu.async_copy(T_hbm.at[idx_v], rows_v, sem).wait()
            pltpu.sync_copy(rows_v, out_X.at[pl.ds(base, CHUNK)])
            return carry

        lax.fori_loop(0, EDGE_CHUNKS, x_body, 0)

        @pl.when(wid < NODE_WORKERS)
        def _node_phase():
            nbase = wid * NODE_PER_W

            def n_body(i, carry):
                base = nbase + i * CHUNK
                pltpu.sync_copy(x_hbm.at[pl.ds(base, CHUNK)], idx_v)
                pltpu.async_copy(Wx_hbm.at[idx_v], rows_v, sem).wait()
                pltpu.sync_copy(rows_v, out_x.at[pl.ds(base, CHUNK)])
                return carry

            lax.fori_loop(0, NODE_CHUNKS, n_body, 0)

    return k(x, A, Xa, Xb, W_x, W_ea, T)


def kernel(x, A, X, W_x, W_ea, W_t1, W_t2):
    T = _fuse_tables(W_t1, W_t2)
    Xa = X[:, 0]
    Xb = X[:, 1]
    x_emb, A_emb, X_emb = _sc_gather(x, A, Xa, Xb, W_x, W_ea, T)
    return (x_emb, A_emb, X_emb)


# speedup vs baseline: 1.5036x; 1.0459x over previous
import functools

import jax
import jax.numpy as jnp
from jax import lax
from jax.experimental import pallas as pl
from jax.experimental.pallas import tpu as pltpu
from jax.experimental.pallas import tpu_sc as plsc

HID = 128
N_NODES = 10000
N_EDGES = 320000

NC, NS = 2, 16
NW = NC * NS
CHUNK = 80
NBUF = 5

EDGE_PER_W = N_EDGES // NW
EDGE_CHUNKS = EDGE_PER_W // CHUNK
EDGE_GROUPS = EDGE_CHUNKS // NBUF
NODE_WORKERS = 25
NODE_PER_W = N_NODES // NODE_WORKERS
NODE_CHUNKS = NODE_PER_W // CHUNK
LANES = 16


def _fuse_tables(W_t1, W_t2):
    def body(w1_ref, w2_ref, out_ref):
        w1 = w1_ref[...]
        w2 = w2_ref[...]
        out_ref[...] = w1[:, None, :] + w2[None, :, :]

    T = pl.pallas_call(
        body,
        out_shape=jax.ShapeDtypeStruct((16, 16, HID), jnp.float32),
    )(W_t1, W_t2)
    return T.reshape(16 * 16, HID)


def _sc_gather(x3, A3, Xa3, Xb3, W_x, W_ea, T):
    mesh = plsc.VectorSubcoreMesh(core_axis_name="c", subcore_axis_name="s")

    @functools.partial(
        pl.kernel,
        out_type=(
            jax.ShapeDtypeStruct((N_NODES, HID), jnp.float32),
            jax.ShapeDtypeStruct((N_EDGES, HID), jnp.float32),
            jax.ShapeDtypeStruct((N_EDGES, HID), jnp.float32),
        ),
        mesh=mesh,
        scratch_types=[
            pltpu.VMEM((EDGE_CHUNKS, CHUNK), jnp.int32),
            pltpu.VMEM((EDGE_CHUNKS, CHUNK), jnp.int32),
            pltpu.VMEM((NBUF, CHUNK, HID), jnp.float32),
            [pltpu.SemaphoreType.DMA] * NBUF,
            [pltpu.SemaphoreType.DMA] * NBUF,
        ],
    )
    def k(x_hbm, A_hbm, Xa_hbm, Xb_hbm, Wx_hbm, Wea_hbm, T_hbm,
          out_x, out_A, out_X, idx_v, idx2_v, rows_v, gsems, wsems):
        wid = lax.axis_index("s") * NC + lax.axis_index("c")
        ebase = wid * EDGE_PER_W

        def pipelined_phase(table_hbm, out_hbm, n_groups):
            def gather(i, b):
                return pltpu.make_async_copy(
                    table_hbm.at[idx_v.at[i]], rows_v.at[b], gsems[b])

            def write(i, b):
                return pltpu.make_async_copy(
                    rows_v.at[b], out_hbm.at[pl.ds(ebase + i * CHUNK, CHUNK)],
                    wsems[b])

            for b in range(NBUF):
                gather(b, b).start()

            def grp(g, carry):
                for b in range(NBUF):
                    i = g * NBUF + b
                    gather(i, b).wait()
                    write(i, b).start()
                    write(i, b).wait()

                    @pl.when(g < n_groups - 1)
                    def _():
                        gather(i + NBUF, b).start()
                return carry

            lax.fori_loop(0, n_groups, grp, 0)

        pltpu.sync_copy(A_hbm.at[wid], idx_v)
        pipelined_phase(Wea_hbm, out_A, EDGE_GROUPS)

        pltpu.sync_copy(Xa_hbm.at[wid], idx_v)
        pltpu.sync_copy(Xb_hbm.at[wid], idx2_v)

        def cidx_row(i, carry):
            for j in range(CHUNK // LANES):
                sl = pl.ds(j * LANES, LANES)
                idx_v[i, sl] = idx_v[i, sl] * 16 + idx2_v[i, sl]
            return carry

        lax.fori_loop(0, EDGE_CHUNKS, cidx_row, 0)
        pipelined_phase(T_hbm, out_X, EDGE_GROUPS)

        @pl.when(wid < NODE_WORKERS)
        def _node_phase():
            pltpu.sync_copy(x_hbm.at[wid], idx_v.at[pl.ds(0, NODE_CHUNKS)])
            nbase = wid * NODE_PER_W

            def n_body(i, carry):
                pltpu.async_copy(
                    Wx_hbm.at[idx_v.at[i]], rows_v.at[0], gsems[0]).wait()
                pltpu.sync_copy(
                    rows_v.at[0],
                    out_x.at[pl.ds(nbase + i * CHUNK, CHUNK)])
                return carry

            lax.fori_loop(0, NODE_CHUNKS, n_body, 0)

    return k(x3, A3, Xa3, Xb3, W_x, W_ea, T)


def kernel(x, A, X, W_x, W_ea, W_t1, W_t2):
    T = _fuse_tables(W_t1, W_t2)
    A3 = A.reshape(NW, EDGE_CHUNKS, CHUNK)
    Xa3 = X[:, 0].reshape(NW, EDGE_CHUNKS, CHUNK)
    Xb3 = X[:, 1].reshape(NW, EDGE_CHUNKS, CHUNK)
    x3 = x.reshape(NODE_WORKERS, NODE_CHUNKS, CHUNK)
    x_emb, A_emb, X_emb = _sc_gather(x3, A3, Xa3, Xb3, W_x, W_ea, T)
    return (x_emb, A_emb, X_emb)


# speedup vs baseline: 1.5057x; 1.0014x over previous
import functools

import jax
import jax.numpy as jnp
from jax import lax
from jax.experimental import pallas as pl
from jax.experimental.pallas import tpu as pltpu
from jax.experimental.pallas import tpu_sc as plsc

HID = 128
N_NODES = 10000
N_EDGES = 320000

NC, NS = 2, 16
NW = NC * NS
CHUNK = 80
NBUF = 5

EDGE_PER_W = N_EDGES // NW
EDGE_CHUNKS = EDGE_PER_W // CHUNK
EDGE_GROUPS = EDGE_CHUNKS // NBUF
NODE_WORKERS = 25
NODE_PER_W = N_NODES // NODE_WORKERS
NODE_CHUNKS = NODE_PER_W // CHUNK
LANES = 16


def _fuse_tables(W_t1, W_t2):
    def body(w1_ref, w2_ref, out_ref):
        w1 = w1_ref[...]
        w2 = w2_ref[...]
        out_ref[...] = w1[:, None, :] + w2[None, :, :]

    T = pl.pallas_call(
        body,
        out_shape=jax.ShapeDtypeStruct((16, 16, HID), jnp.float32),
    )(W_t1, W_t2)
    return T.reshape(16 * 16, HID)


def _sc_gather(x3, A3, Xa3, Xb3, W_x, W_ea, T):
    mesh = plsc.VectorSubcoreMesh(core_axis_name="c", subcore_axis_name="s")

    @functools.partial(
        pl.kernel,
        out_type=(
            jax.ShapeDtypeStruct((N_NODES, HID), jnp.float32),
            jax.ShapeDtypeStruct((N_EDGES, HID), jnp.float32),
            jax.ShapeDtypeStruct((N_EDGES, HID), jnp.float32),
        ),
        mesh=mesh,
        scratch_types=[
            pltpu.VMEM((EDGE_CHUNKS, CHUNK), jnp.int32),
            pltpu.VMEM((EDGE_CHUNKS, CHUNK), jnp.int32),
            pltpu.VMEM((NBUF, CHUNK, HID), jnp.float32),
            [pltpu.SemaphoreType.DMA] * NBUF,
            [pltpu.SemaphoreType.DMA] * NBUF,
        ],
    )
    def k(x_hbm, A_hbm, Xa_hbm, Xb_hbm, Wx_hbm, Wea_hbm, T_hbm,
          out_x, out_A, out_X, idx_v, idx2_v, rows_v, gsems, wsems):
        wid = lax.axis_index("s") * NC + lax.axis_index("c")
        ebase = wid * EDGE_PER_W

        def pipelined_phase(table_hbm, out_hbm, n_chunks, base):
            D = 2

            def gather(i, b):
                return pltpu.make_async_copy(
                    table_hbm.at[idx_v.at[i]], rows_v.at[b], gsems[b])

            def write(i, b):
                return pltpu.make_async_copy(
                    rows_v.at[b], out_hbm.at[pl.ds(base + i * CHUNK, CHUNK)],
                    wsems[b])

            for b in range(D):
                gather(b, b).start()

            def grp(g, carry):
                for b in range(NBUF):
                    i = g * NBUF + b
                    gather(i, b).wait()
                    write(i, b).start()
                    j = i + D
                    bj = (b + D) % NBUF

                    @pl.when(jnp.logical_and(j >= NBUF, j < n_chunks))
                    def _():
                        write(j - NBUF, bj).wait()

                    @pl.when(j < n_chunks)
                    def _():
                        gather(j, bj).start()
                return carry

            lax.fori_loop(0, n_chunks // NBUF, grp, 0)
            for t in range(NBUF):
                write(n_chunks - NBUF + t, (n_chunks - NBUF + t) % NBUF).wait()

        pltpu.sync_copy(A_hbm.at[wid], idx_v)
        pipelined_phase(Wea_hbm, out_A, EDGE_CHUNKS, ebase)

        pltpu.sync_copy(Xa_hbm.at[wid], idx_v)
        pltpu.sync_copy(Xb_hbm.at[wid], idx2_v)

        def cidx_row(i, carry):
            for j in range(CHUNK // LANES):
                sl = pl.ds(j * LANES, LANES)
                idx_v[i, sl] = idx_v[i, sl] * 16 + idx2_v[i, sl]
            return carry

        lax.fori_loop(0, EDGE_CHUNKS, cidx_row, 0)
        pipelined_phase(T_hbm, out_X, EDGE_CHUNKS, ebase)

        @pl.when(wid < NODE_WORKERS)
        def _node_phase():
            pltpu.sync_copy(x_hbm.at[wid], idx_v.at[pl.ds(0, NODE_CHUNKS)])
            pipelined_phase(Wx_hbm, out_x, NODE_CHUNKS, wid * NODE_PER_W)

    return k(x3, A3, Xa3, Xb3, W_x, W_ea, T)


def kernel(x, A, X, W_x, W_ea, W_t1, W_t2):
    T = _fuse_tables(W_t1, W_t2)
    A3 = A.reshape(NW, EDGE_CHUNKS, CHUNK)
    Xa3 = X[:, 0].reshape(NW, EDGE_CHUNKS, CHUNK)
    Xb3 = X[:, 1].reshape(NW, EDGE_CHUNKS, CHUNK)
    x3 = x.reshape(NODE_WORKERS, NODE_CHUNKS, CHUNK)
    x_emb, A_emb, X_emb = _sc_gather(x3, A3, Xa3, Xb3, W_x, W_ea, T)
    return (x_emb, A_emb, X_emb)


# speedup vs baseline: 2.2863x; 1.5184x over previous
import functools

import jax
import jax.numpy as jnp
from jax import lax
from jax.experimental import pallas as pl
from jax.experimental.pallas import tpu as pltpu
from jax.experimental.pallas import tpu_sc as plsc

HID = 128
N_NODES = 10000
N_EDGES = 320000

NC, NS = 2, 16
NW = NC * NS
CHUNK = 80
NBUF = 5

EDGE_PER_W = N_EDGES // NW
EDGE_CHUNKS = EDGE_PER_W // CHUNK
NODE_WORKERS = 25
NODE_PER_W = N_NODES // NODE_WORKERS
NODE_CHUNKS = NODE_PER_W // CHUNK
LANES = 16


def _bcast_lane(vec, lane):
    idx = jnp.full((LANES,), lane, jnp.int32)
    dnums = lax.GatherDimensionNumbers(
        offset_dims=(), collapsed_slice_dims=(0,), start_index_map=(0,))
    return lax.gather(vec, idx[:, None], dnums, (1,),
                      mode=lax.GatherScatterMode.PROMISE_IN_BOUNDS)


def _fuse_tables(W_t1, W_t2):
    def body(w1_ref, w2_ref, out_ref):
        w1 = w1_ref[...]
        w2 = w2_ref[...]
        out_ref[...] = w1[:, None, :] + w2[None, :, :]

    T = pl.pallas_call(
        body,
        out_shape=jax.ShapeDtypeStruct((16, 16, HID), jnp.float32),
    )(W_t1, W_t2)
    return T.reshape(16 * 16, HID)


def _sc_gather(x3, A3, Xa3, Xb3, W_x, W_ea, T):
    mesh = plsc.VectorSubcoreMesh(core_axis_name="c", subcore_axis_name="s")

    @functools.partial(
        pl.kernel,
        out_type=(
            jax.ShapeDtypeStruct((N_NODES, HID), jnp.float32),
            jax.ShapeDtypeStruct((N_EDGES, HID), jnp.float32),
            jax.ShapeDtypeStruct((N_EDGES, HID), jnp.float32),
        ),
        mesh=mesh,
        compiler_params=pltpu.CompilerParams(needs_layout_passes=False),
        scratch_types=[
            pltpu.VMEM((EDGE_CHUNKS, CHUNK), jnp.int32),
            pltpu.VMEM((EDGE_CHUNKS, CHUNK), jnp.int32),
            pltpu.VMEM((32, HID), jnp.float32),
            pltpu.VMEM((16, HID), jnp.float32),
            pltpu.VMEM((256, HID), jnp.float32),
            pltpu.VMEM((NBUF, CHUNK, HID), jnp.float32),
            pltpu.SemaphoreType.DMA((NBUF,)),
        ],
    )
    def k(x_hbm, A_hbm, Xa_hbm, Xb_hbm, Wx_hbm, Wea_hbm, T_hbm,
          out_x, out_A, out_X, idx_v, idx2_v, wx_v, wea_v, t_v, rows_v, wsems):
        wid = lax.axis_index("s") * NC + lax.axis_index("c")
        ebase = wid * EDGE_PER_W
        lanes = lax.iota(jnp.int32, LANES)

        pltpu.sync_copy(Wx_hbm, wx_v)
        pltpu.sync_copy(Wea_hbm, wea_v)
        pltpu.sync_copy(T_hbm, t_v)

        def gen_phase(tab_v, out_hbm, n_chunks, base):
            def write(i, b):
                return pltpu.make_async_copy(
                    rows_v.at[b], out_hbm.at[pl.ds(base + i * CHUNK, CHUNK)],
                    wsems.at[b])

            def chunk(i, carry):
                b = lax.rem(i, NBUF)

                @pl.when(i >= NBUF)
                def _():
                    write(i - NBUF, b).wait()

                for g in range(CHUNK // LANES):
                    va = idx_v[i, pl.ds(g * LANES, LANES)]
                    for r in range(LANES):
                        row = g * LANES + r
                        bc = _bcast_lane(va, r)
                        for c in range(HID // LANES):
                            vals = plsc.load_gather(
                                tab_v, [bc, lanes + (c * LANES)])
                            rows_v[b, row, pl.ds(c * LANES, LANES)] = vals

                write(i, b).start()
                return carry

            lax.fori_loop(0, n_chunks, chunk, 0)
            for t in range(min(NBUF, n_chunks)):
                i = n_chunks - min(NBUF, n_chunks) + t
                write(i, i % NBUF).wait()

        pltpu.sync_copy(A_hbm.at[wid], idx_v)
        gen_phase(wea_v, out_A, EDGE_CHUNKS, ebase)

        pltpu.sync_copy(Xa_hbm.at[wid], idx_v)
        pltpu.sync_copy(Xb_hbm.at[wid], idx2_v)

        def cidx_row(i, carry):
            for j in range(CHUNK // LANES):
                sl = pl.ds(j * LANES, LANES)
                idx_v[i, sl] = idx_v[i, sl] * 16 + idx2_v[i, sl]
            return carry

        lax.fori_loop(0, EDGE_CHUNKS, cidx_row, 0)
        gen_phase(t_v, out_X, EDGE_CHUNKS, ebase)

        @pl.when(wid < NODE_WORKERS)
        def _node_phase():
            pltpu.sync_copy(x_hbm.at[wid], idx_v.at[pl.ds(0, NODE_CHUNKS)])
            gen_phase(wx_v, out_x, NODE_CHUNKS, wid * NODE_PER_W)

    return k(x3, A3, Xa3, Xb3, W_x, W_ea, T)


def kernel(x, A, X, W_x, W_ea, W_t1, W_t2):
    T = _fuse_tables(W_t1, W_t2)
    A3 = A.reshape(NW, EDGE_CHUNKS, CHUNK)
    Xa3 = X[:, 0].reshape(NW, EDGE_CHUNKS, CHUNK)
    Xb3 = X[:, 1].reshape(NW, EDGE_CHUNKS, CHUNK)
    x3 = x.reshape(NODE_WORKERS, NODE_CHUNKS, CHUNK)
    x_emb, A_emb, X_emb = _sc_gather(x3, A3, Xa3, Xb3, W_x, W_ea, T)
    return (x_emb, A_emb, X_emb)


# speedup vs baseline: 8.6143x; 3.7678x over previous
import functools

import jax
import jax.numpy as jnp
from jax import lax
from jax.experimental import pallas as pl
from jax.experimental.pallas import tpu as pltpu
from jax.experimental.pallas import tpu_sc as plsc

HID = 128
N_NODES = 10000
N_EDGES = 320000

NC, NS = 2, 16
NW = NC * NS
CHUNK = 80
NBUF = 5

EDGE_PER_W = N_EDGES // NW
EDGE_CHUNKS = EDGE_PER_W // CHUNK
NODE_WORKERS = 25
NODE_PER_W = N_NODES // NODE_WORKERS
NODE_CHUNKS = NODE_PER_W // CHUNK
LANES = 16


def _bcast_lane(vec, lane):
    idx = jnp.full((LANES,), lane, jnp.int32)
    dnums = lax.GatherDimensionNumbers(
        offset_dims=(), collapsed_slice_dims=(0,), start_index_map=(0,))
    return lax.gather(vec, idx[:, None], dnums, (1,),
                      mode=lax.GatherScatterMode.PROMISE_IN_BOUNDS)


def _fuse_tables(W_t1, W_t2):
    def body(w1_ref, w2_ref, out_ref):
        w1 = w1_ref[...]
        w2 = w2_ref[...]
        out_ref[...] = w1[:, None, :] + w2[None, :, :]

    T = pl.pallas_call(
        body,
        out_shape=jax.ShapeDtypeStruct((16, 16, HID), jnp.float32),
    )(W_t1, W_t2)
    return T.reshape(16 * 16, HID)


def _sc_gather(x3, A3, Xa3, Xb3, W_x, W_ea, T):
    mesh = plsc.VectorSubcoreMesh(core_axis_name="c", subcore_axis_name="s")

    @functools.partial(
        pl.kernel,
        out_type=(
            jax.ShapeDtypeStruct((N_NODES, HID), jnp.float32),
            jax.ShapeDtypeStruct((N_EDGES, HID), jnp.float32),
            jax.ShapeDtypeStruct((N_EDGES, HID), jnp.float32),
        ),
        mesh=mesh,
        compiler_params=pltpu.CompilerParams(needs_layout_passes=False),
        scratch_types=[
            pltpu.VMEM((EDGE_CHUNKS, CHUNK), jnp.int32),
            pltpu.VMEM((EDGE_CHUNKS, CHUNK), jnp.int32),
            pltpu.VMEM((32, HID), jnp.float32),
            pltpu.VMEM((16, HID), jnp.float32),
            pltpu.VMEM((256, HID), jnp.float32),
            pltpu.VMEM((NBUF, CHUNK, HID), jnp.float32),
            pltpu.SemaphoreType.DMA((NBUF,)),
        ],
    )
    def k(x_hbm, A_hbm, Xa_hbm, Xb_hbm, Wx_hbm, Wea_hbm, T_hbm,
          out_x, out_A, out_X, idx_v, idx2_v, wx_v, wea_v, t_v, rows_v, wsems):
        wid = lax.axis_index("s") * NC + lax.axis_index("c")
        ebase = wid * EDGE_PER_W
        lanes = lax.iota(jnp.int32, LANES)

        pltpu.sync_copy(Wx_hbm, wx_v)
        pltpu.sync_copy(Wea_hbm, wea_v)
        pltpu.sync_copy(T_hbm, t_v)

        def gen_phase(tab_v, out_hbm, n_chunks, base):
            def write(i, b):
                return pltpu.make_async_copy(
                    rows_v.at[b], out_hbm.at[pl.ds(base + i * CHUNK, CHUNK)],
                    wsems.at[b])

            def chunk(i, carry):
                b = lax.rem(i, NBUF)

                @pl.when(i >= NBUF)
                def _():
                    write(i - NBUF, b).wait()

                for g in range(CHUNK // LANES):
                    va = idx_v[i, pl.ds(g * LANES, LANES)]
                    for r in range(LANES):
                        row = g * LANES + r
                        bc = _bcast_lane(va, r)
                        vals = [plsc.load_gather(tab_v, [bc, lanes + (c * LANES)])
                                for c in range(HID // LANES)]
                        for c in range(HID // LANES):
                            rows_v[b, row, pl.ds(c * LANES, LANES)] = vals[c]

                write(i, b).start()
                return carry

            lax.fori_loop(0, n_chunks, chunk, 0)
            for t in range(min(NBUF, n_chunks)):
                i = n_chunks - min(NBUF, n_chunks) + t
                write(i, i % NBUF).wait()

        pltpu.sync_copy(A_hbm.at[wid], idx_v)
        gen_phase(wea_v, out_A, EDGE_CHUNKS, ebase)

        pltpu.sync_copy(Xa_hbm.at[wid], idx_v)
        pltpu.sync_copy(Xb_hbm.at[wid], idx2_v)

        def cidx_row(i, carry):
            for j in range(CHUNK // LANES):
                sl = pl.ds(j * LANES, LANES)
                idx_v[i, sl] = idx_v[i, sl] * 16 + idx2_v[i, sl]
            return carry

        lax.fori_loop(0, EDGE_CHUNKS, cidx_row, 0)
        gen_phase(t_v, out_X, EDGE_CHUNKS, ebase)

        @pl.when(wid < NODE_WORKERS)
        def _node_phase():
            pltpu.sync_copy(x_hbm.at[wid], idx_v.at[pl.ds(0, NODE_CHUNKS)])
            gen_phase(wx_v, out_x, NODE_CHUNKS, wid * NODE_PER_W)

    return k(x3, A3, Xa3, Xb3, W_x, W_ea, T)


def kernel(x, A, X, W_x, W_ea, W_t1, W_t2):
    T = _fuse_tables(W_t1, W_t2)
    A3 = A.reshape(NW, EDGE_CHUNKS, CHUNK)
    Xa3 = X[:, 0].reshape(NW, EDGE_CHUNKS, CHUNK)
    Xb3 = X[:, 1].reshape(NW, EDGE_CHUNKS, CHUNK)
    x3 = x.reshape(NODE_WORKERS, NODE_CHUNKS, CHUNK)
    x_emb, A_emb, X_emb = _sc_gather(x3, A3, Xa3, Xb3, W_x, W_ea, T)
    return (x_emb, A_emb, X_emb)


# speedup vs baseline: 10.3867x; 1.2058x over previous
import functools

import jax
import jax.numpy as jnp
from jax import lax
from jax.experimental import pallas as pl
from jax.experimental.pallas import tpu as pltpu
from jax.experimental.pallas import tpu_sc as plsc

HID = 128
N_NODES = 10000
N_EDGES = 320000

NC, NS = 2, 16
NW = NC * NS
CHUNK = 80
NBUF = 5

EDGE_PER_W = N_EDGES // NW
EDGE_CHUNKS = EDGE_PER_W // CHUNK
NODE_WORKERS = 25
NODE_PER_W = N_NODES // NODE_WORKERS
NODE_CHUNKS = NODE_PER_W // CHUNK
LANES = 16

TC_ROWS = 2000
TC_GRID = N_EDGES // TC_ROWS


def _bcast_lane(vec, lane):
    idx = jnp.full((LANES,), lane, jnp.int32)
    dnums = lax.GatherDimensionNumbers(
        offset_dims=(), collapsed_slice_dims=(0,), start_index_map=(0,))
    return lax.gather(vec, idx[:, None], dnums, (1,),
                      mode=lax.GatherScatterMode.PROMISE_IN_BOUNDS)


def _tc_xemb(X, W_t1, W_t2):
    Xa = X[:, 0].reshape(TC_GRID, 1, TC_ROWS)
    Xb = X[:, 1].reshape(TC_GRID, 1, TC_ROWS)

    def body(xa_ref, xb_ref, w1_ref, w2_ref, out_ref):
        a = xa_ref[0, 0, :]
        b = xb_ref[0, 0, :]
        iot = lax.broadcasted_iota(jnp.int32, (TC_ROWS, 16), 1)
        oh_a = (a[:, None] == iot).astype(jnp.float32)
        oh_b = (b[:, None] == iot).astype(jnp.float32)
        out_ref[0] = (jnp.dot(oh_a, w1_ref[...], preferred_element_type=jnp.float32)
                      + jnp.dot(oh_b, w2_ref[...], preferred_element_type=jnp.float32))

    out = pl.pallas_call(
        body,
        grid=(TC_GRID,),
        in_specs=[
            pl.BlockSpec((1, 1, TC_ROWS), lambda i: (i, 0, 0)),
            pl.BlockSpec((1, 1, TC_ROWS), lambda i: (i, 0, 0)),
            pl.BlockSpec((16, HID), lambda i: (0, 0)),
            pl.BlockSpec((16, HID), lambda i: (0, 0)),
        ],
        out_specs=pl.BlockSpec((1, TC_ROWS, HID), lambda i: (i, 0, 0)),
        out_shape=jax.ShapeDtypeStruct((TC_GRID, TC_ROWS, HID), jnp.float32),
    )(Xa, Xb, W_t1, W_t2)
    return out.reshape(N_EDGES, HID)


def _sc_gather(x3, A3, W_x, W_ea):
    mesh = plsc.VectorSubcoreMesh(core_axis_name="c", subcore_axis_name="s")

    @functools.partial(
        pl.kernel,
        out_type=(
            jax.ShapeDtypeStruct((N_NODES, HID), jnp.float32),
            jax.ShapeDtypeStruct((N_EDGES, HID), jnp.float32),
        ),
        mesh=mesh,
        compiler_params=pltpu.CompilerParams(needs_layout_passes=False),
        scratch_types=[
            pltpu.VMEM((EDGE_CHUNKS, CHUNK), jnp.int32),
            pltpu.VMEM((32, HID), jnp.float32),
            pltpu.VMEM((16, HID), jnp.float32),
            pltpu.VMEM((NBUF, CHUNK, HID), jnp.float32),
            pltpu.SemaphoreType.DMA((NBUF,)),
        ],
    )
    def k(x_hbm, A_hbm, Wx_hbm, Wea_hbm,
          out_x, out_A, idx_v, wx_v, wea_v, rows_v, wsems):
        wid = lax.axis_index("s") * NC + lax.axis_index("c")
        ebase = wid * EDGE_PER_W
        lanes = lax.iota(jnp.int32, LANES)

        pltpu.sync_copy(Wx_hbm, wx_v)
        pltpu.sync_copy(Wea_hbm, wea_v)

        def gen_phase(tab_v, out_hbm, n_chunks, base):
            def write(i, b):
                return pltpu.make_async_copy(
                    rows_v.at[b], out_hbm.at[pl.ds(base + i * CHUNK, CHUNK)],
                    wsems.at[b])

            def chunk(i, carry):
                b = lax.rem(i, NBUF)

                @pl.when(i >= NBUF)
                def _():
                    write(i - NBUF, b).wait()

                for g in range(CHUNK // LANES):
                    va = idx_v[i, pl.ds(g * LANES, LANES)]
                    for r in range(LANES):
                        row = g * LANES + r
                        bc = _bcast_lane(va, r)
                        vals = [plsc.load_gather(tab_v, [bc, lanes + (c * LANES)])
                                for c in range(HID // LANES)]
                        for c in range(HID // LANES):
                            rows_v[b, row, pl.ds(c * LANES, LANES)] = vals[c]

                write(i, b).start()
                return carry

            lax.fori_loop(0, n_chunks, chunk, 0)
            for t in range(min(NBUF, n_chunks)):
                i = n_chunks - min(NBUF, n_chunks) + t
                write(i, i % NBUF).wait()

        pltpu.sync_copy(A_hbm.at[wid], idx_v)
        gen_phase(wea_v, out_A, EDGE_CHUNKS, ebase)

        @pl.when(wid < NODE_WORKERS)
        def _node_phase():
            pltpu.sync_copy(x_hbm.at[wid], idx_v.at[pl.ds(0, NODE_CHUNKS)])
            gen_phase(wx_v, out_x, NODE_CHUNKS, wid * NODE_PER_W)

    return k(x3, A3, W_x, W_ea)


def kernel(x, A, X, W_x, W_ea, W_t1, W_t2):
    A3 = A.reshape(NW, EDGE_CHUNKS, CHUNK)
    x3 = x.reshape(NODE_WORKERS, NODE_CHUNKS, CHUNK)
    X_emb = _tc_xemb(X, W_t1, W_t2)
    x_emb, A_emb = _sc_gather(x3, A3, W_x, W_ea)
    return (x_emb, A_emb, X_emb)
